# Initial kernel scaffold; baseline (speedup 1.0000x reference)
import jax
import jax.numpy as jnp
from jax.experimental import pallas as pl


def kernel(ents, rels_tab, se0, se1, se2, sr0, sr1, sr2, heads, rels, tails, sources, heads_bad, rels_bad, tails_bad, sources_bad):
    raise NotImplementedError("write your pallas kernel here")


# speedup vs baseline: 1.4952x; 1.4952x over previous
import functools

import jax
import jax.numpy as jnp
from jax import lax
from jax.experimental import pallas as pl
from jax.experimental.pallas import tpu as pltpu
from jax.experimental.pallas import tpu_sc as plsc

NC, NS = 2, 16
NW = NC * NS
B2 = 8192
DIM = 128
CHUNK = 128
PER_W = B2 // NW
NCHUNK = PER_W // CHUNK
NGATHER = 12


def _sc_gather_all(tables, idxs):
    mesh = plsc.VectorSubcoreMesh(core_axis_name="c", subcore_axis_name="s")
    out_type = [jax.ShapeDtypeStruct((B2, DIM), jnp.float32)] * NGATHER

    @functools.partial(
        pl.kernel,
        mesh=mesh,
        out_type=out_type,
        scratch_types=[
            pltpu.VMEM((CHUNK,), jnp.int32),
            pltpu.VMEM((CHUNK, DIM), jnp.float32),
            pltpu.SemaphoreType.DMA,
        ],
    )
    def k(*refs):
        t_refs = refs[:NGATHER]
        i_refs = refs[NGATHER:2 * NGATHER]
        o_refs = refs[2 * NGATHER:3 * NGATHER]
        idx_v, rows_v, sem = refs[3 * NGATHER:]
        wid = lax.axis_index("s") * NC + lax.axis_index("c")
        base0 = wid * PER_W
        for g in range(NGATHER):
            for c in range(NCHUNK):
                base = base0 + c * CHUNK
                pltpu.sync_copy(i_refs[g].at[pl.ds(base, CHUNK)], idx_v)
                pltpu.async_copy(t_refs[g].at[idx_v], rows_v, sem).wait()
                pltpu.sync_copy(rows_v, o_refs[g].at[pl.ds(base, CHUNK)])

    return k(*tables, *idxs)


def _tc_score(g):
    blk = 512

    def body(h, sh0, sh1, sh2, t, st0, st1, st2, r, sq0, sq1, sq2, o):
        def nrm(x):
            n = jnp.sqrt(jnp.sum(x * x, axis=1, keepdims=True))
            return x / jnp.maximum(n, 1e-12)

        hv = nrm(h[...]) + nrm(sh0[...]) + nrm(sh1[...]) + nrm(sh2[...])
        tv = nrm(t[...]) + nrm(st0[...]) + nrm(st1[...]) + nrm(st2[...])
        rv = nrm(r[...]) + nrm(sq0[...]) + nrm(sq1[...]) + nrm(sq2[...])
        d = nrm(hv) + nrm(rv) - nrm(tv)
        o[...] = jnp.sqrt(jnp.sum(d * d, axis=1, keepdims=True))

    in_spec = pl.BlockSpec((blk, DIM), lambda i: (i, 0))
    out_spec = pl.BlockSpec((blk, 1), lambda i: (i, 0))
    return pl.pallas_call(
        body,
        grid=(B2 // blk,),
        in_specs=[in_spec] * NGATHER,
        out_specs=out_spec,
        out_shape=jax.ShapeDtypeStruct((B2, 1), jnp.float32),
    )(*g)


def kernel(ents, rels_tab, se0, se1, se2, sr0, sr1, sr2,
           heads, rels, tails, sources,
           heads_bad, rels_bad, tails_bad, sources_bad):
    ah = jnp.concatenate([heads, heads_bad]).astype(jnp.int32)
    ar = jnp.concatenate([rels, rels_bad]).astype(jnp.int32)
    at = jnp.concatenate([tails, tails_bad]).astype(jnp.int32)
    asrc = jnp.concatenate([sources, sources_bad])
    z = jnp.zeros((), jnp.int32)
    masks = [asrc == (j + 2) for j in range(3)]

    tables = ([ents] + [se0, se1, se2]
              + [ents] + [se0, se1, se2]
              + [rels_tab] + [sr0, sr1, sr2])
    idxs = ([ah] + [jnp.where(m, ah, z) for m in masks]
            + [at] + [jnp.where(m, at, z) for m in masks]
            + [ar] + [jnp.where(m, ar, z) for m in masks])

    g = _sc_gather_all(tables, idxs)
    s = _tc_score(g)[:, 0]
    return (s[:4096], s[4096:])


# speedup vs baseline: 1.6261x; 1.0876x over previous
import functools

import jax
import jax.numpy as jnp
from jax import lax
from jax.experimental import pallas as pl
from jax.experimental.pallas import tpu as pltpu
from jax.experimental.pallas import tpu_sc as plsc

NC, NS = 2, 16
NW = NC * NS
B2 = 8192
DIM = 128
CHUNK = 128
PER_W = B2 // NW
NCHUNK = PER_W // CHUNK
NGATHER = 12


def _sc_gather_all(tables, idxs):
    mesh = plsc.VectorSubcoreMesh(core_axis_name="c", subcore_axis_name="s")
    out_type = [jax.ShapeDtypeStruct((B2, DIM), jnp.float32)] * NGATHER
    nslots = NGATHER * NCHUNK

    @functools.partial(
        pl.kernel,
        mesh=mesh,
        out_type=out_type,
        scratch_types=(
            [pltpu.VMEM((PER_W,), jnp.int32)] * NGATHER
            + [pltpu.VMEM((CHUNK, DIM), jnp.float32)] * 2
            + [pltpu.SemaphoreType.DMA] * 5
        ),
    )
    def k(*refs):
        t_refs = refs[:NGATHER]
        i_refs = refs[NGATHER:2 * NGATHER]
        o_refs = refs[2 * NGATHER:3 * NGATHER]
        idx_v = refs[3 * NGATHER:4 * NGATHER]
        rows_v = refs[4 * NGATHER:4 * NGATHER + 2]
        sem_i = refs[4 * NGATHER + 2]
        sem_g = refs[4 * NGATHER + 3:4 * NGATHER + 5]
        sem_w = refs[4 * NGATHER + 5:4 * NGATHER + 7]
        wid = lax.axis_index("s") * NC + lax.axis_index("c")
        base0 = wid * PER_W

        pf = [pltpu.make_async_copy(i_refs[g].at[pl.ds(base0, PER_W)],
                                    idx_v[g], sem_i) for g in range(NGATHER)]
        for d in pf:
            d.start()
        for d in pf:
            d.wait()

        def slot(kk):
            g, c = kk // NCHUNK, kk % NCHUNK
            return g, c * CHUNK

        gd = [None, None]
        wd = [None, None]
        for kk in range(nslots):
            s = kk % 2
            if wd[s] is not None:
                wd[s].wait()
            g, off = slot(kk)
            gd[s] = pltpu.make_async_copy(
                t_refs[g].at[idx_v[g].at[pl.ds(off, CHUNK)]],
                rows_v[s], sem_g[s])
            gd[s].start()
            if kk >= 1:
                p = (kk - 1) % 2
                gd[p].wait()
                pg, poff = slot(kk - 1)
                wd[p] = pltpu.make_async_copy(
                    rows_v[p], o_refs[pg].at[pl.ds(base0 + poff, CHUNK)],
                    sem_w[p])
                wd[p].start()
        s_last = (nslots - 1) % 2
        gd[s_last].wait()
        pg, poff = slot(nslots - 1)
        pltpu.sync_copy(rows_v[s_last], o_refs[pg].at[pl.ds(base0 + poff, CHUNK)])
        wd[(nslots - 2) % 2].wait()

    return k(*tables, *idxs)


def _tc_score(g):
    blk = 512

    def body(h, sh0, sh1, sh2, t, st0, st1, st2, r, sq0, sq1, sq2, o):
        def nrm(x):
            n = jnp.sqrt(jnp.sum(x * x, axis=1, keepdims=True))
            return x / jnp.maximum(n, 1e-12)

        hv = nrm(h[...]) + nrm(sh0[...]) + nrm(sh1[...]) + nrm(sh2[...])
        tv = nrm(t[...]) + nrm(st0[...]) + nrm(st1[...]) + nrm(st2[...])
        rv = nrm(r[...]) + nrm(sq0[...]) + nrm(sq1[...]) + nrm(sq2[...])
        d = nrm(hv) + nrm(rv) - nrm(tv)
        o[...] = jnp.sqrt(jnp.sum(d * d, axis=1, keepdims=True))

    in_spec = pl.BlockSpec((blk, DIM), lambda i: (i, 0))
    out_spec = pl.BlockSpec((blk, 1), lambda i: (i, 0))
    return pl.pallas_call(
        body,
        grid=(B2 // blk,),
        in_specs=[in_spec] * NGATHER,
        out_specs=out_spec,
        out_shape=jax.ShapeDtypeStruct((B2, 1), jnp.float32),
    )(*g)


def kernel(ents, rels_tab, se0, se1, se2, sr0, sr1, sr2,
           heads, rels, tails, sources,
           heads_bad, rels_bad, tails_bad, sources_bad):
    ah = jnp.concatenate([heads, heads_bad]).astype(jnp.int32)
    ar = jnp.concatenate([rels, rels_bad]).astype(jnp.int32)
    at = jnp.concatenate([tails, tails_bad]).astype(jnp.int32)
    asrc = jnp.concatenate([sources, sources_bad])
    z = jnp.zeros((), jnp.int32)
    masks = [asrc == (j + 2) for j in range(3)]

    tables = ([ents] + [se0, se1, se2]
              + [ents] + [se0, se1, se2]
              + [rels_tab] + [sr0, sr1, sr2])
    idxs = ([ah] + [jnp.where(m, ah, z) for m in masks]
            + [at] + [jnp.where(m, at, z) for m in masks]
            + [ar] + [jnp.where(m, ar, z) for m in masks])

    g = _sc_gather_all(tables, idxs)
    s = _tc_score(g)[:, 0]
    return (s[:4096], s[4096:])
